# baseline (speedup 1.0000x reference)
import jax
import jax.numpy as jnp
from jax.experimental import pallas as pl

_NUM_FIELDS = 26
_DEPTH = 1000
_BR = 64


def _onehot_block(fv_ref, out_ref):
    fv = fv_ref[...]
    col = jax.lax.broadcasted_iota(jnp.int32, (_BR, _NUM_FIELDS, _DEPTH), 2)
    out_ref[...] = (col == fv[:, :, None]).astype(jnp.float32)


def kernel(feature_value):
    batch = feature_value.shape[0]
    out3 = pl.pallas_call(
        _onehot_block,
        grid=(batch // _BR,),
        in_specs=[pl.BlockSpec((_BR, _NUM_FIELDS), lambda i: (i, 0))],
        out_specs=pl.BlockSpec((_BR, _NUM_FIELDS, _DEPTH), lambda i: (i, 0, 0)),
        out_shape=jax.ShapeDtypeStruct((batch, _NUM_FIELDS, _DEPTH), jnp.float32),
    )(feature_value)
    return out3.reshape(batch, _NUM_FIELDS * _DEPTH)


# speedup vs baseline: 1.2006x; 1.2006x over previous
import jax
import jax.numpy as jnp
from jax.experimental import pallas as pl

_NUM_FIELDS = 26
_DEPTH = 1000
_BR = 128


def _onehot_block(fv_ref, out_ref):
    fv = fv_ref[...]
    iota = jax.lax.broadcasted_iota(jnp.int32, (_BR, _DEPTH), 1)
    for f in range(_NUM_FIELDS):
        out_ref[:, f * _DEPTH:(f + 1) * _DEPTH] = (
            iota == fv[:, f:f + 1]).astype(jnp.float32)


def kernel(feature_value):
    batch = feature_value.shape[0]
    width = _NUM_FIELDS * _DEPTH
    return pl.pallas_call(
        _onehot_block,
        grid=(batch // _BR,),
        in_specs=[pl.BlockSpec((_BR, _NUM_FIELDS), lambda i: (i, 0))],
        out_specs=pl.BlockSpec((_BR, width), lambda i: (i, 0)),
        out_shape=jax.ShapeDtypeStruct((batch, width), jnp.float32),
    )(feature_value)
